# Initial kernel scaffold; baseline (speedup 1.0000x reference)
import jax
import jax.numpy as jnp
from jax.experimental import pallas as pl


def kernel(input_hits, params):
    raise NotImplementedError("write your pallas kernel here")


# speedup vs baseline: 5.5730x; 5.5730x over previous
import jax
import jax.numpy as jnp
from jax.experimental import pallas as pl

F32 = jnp.float32
_KNN = 3
_BIG = 1e30


def _knn_idx(s, sq, block_rows=200):
    n = s.shape[0]
    br = block_rows if n % block_rows == 0 else n

    def body(s_ref, sq_ref, st_ref, sqt_ref, idx_ref):
        dot3 = jnp.dot(s_ref[...], st_ref[...], preferred_element_type=F32)
        d2 = (sq_ref[...] + sqt_ref[...]) - 2.0 * dot3
        iota = jax.lax.broadcasted_iota(jnp.int32, d2.shape, 1)
        idxs = []
        for k in range(_KNN):
            m = jnp.min(d2, axis=1, keepdims=True)
            idx = jnp.min(jnp.where(d2 == m, iota, n), axis=1, keepdims=True)
            idxs.append(idx)
            if k < _KNN - 1:
                d2 = jnp.where(iota == idx, _BIG, d2)
        idx_ref[...] = jnp.concatenate(idxs, axis=1)

    return pl.pallas_call(
        body, grid=(n // br,),
        in_specs=[pl.BlockSpec((br, 3), lambda i: (i, 0)),
                  pl.BlockSpec((br, 1), lambda i: (i, 0)),
                  pl.BlockSpec((3, n), lambda i: (0, 0)),
                  pl.BlockSpec((1, n), lambda i: (0, 0))],
        out_specs=pl.BlockSpec((br, _KNN), lambda i: (i, 0)),
        out_shape=jax.ShapeDtypeStruct((n, _KNN), jnp.int32),
    )(s, sq, s.T, sq.T)


def _linear(p, x):
    return x @ p["W"].T + p["b"]


def _gravnet_conv(p, x):
    s = _linear(p["lin_s"], x)
    h = _linear(p["lin_h"], x)
    sq = jnp.sum(s * s, axis=1)
    idx = _knn_idx(s, sq.reshape(-1, 1))
    s_nb = jnp.take(s, idx, axis=0)
    dist2 = jnp.sum((s[:, None, :] - s_nb) ** 2, axis=-1)
    w = jnp.exp(-10.0 * dist2)
    msg = jnp.take(h, idx, axis=0) * w[..., None]
    agg = jnp.concatenate([jnp.mean(msg, axis=1), jnp.max(msg, axis=1)],
                          axis=-1)
    return x @ p["Wo1"].T + _linear(p["lin_out2"], agg)


def _block(p, x):
    x = x.reshape(x.shape[0], -1)
    x = _linear(p["d1"], x)
    x = jax.nn.relu(_linear(p["d2"], x))
    x = jax.nn.relu(_linear(p["d3"], x))
    for m in ("mp1", "mp2", "mp3", "mp4", "mp5", "mp6"):
        x = _gravnet_conv(p[m], x)
    x = jax.nn.relu(_linear(p["d4"], x))
    x = jax.nn.relu(_linear(p["d5"], x))
    x = jax.nn.relu(_linear(p["d6"], x))
    return x


def kernel(input_hits, params):
    x1 = _block(params["b1"], input_hits)
    x2 = _block(params["b2"], x1)
    x3 = _block(params["b3"], x2)
    x = jnp.concatenate([x1, x2, x3], axis=1)
    x = jax.nn.relu(_linear(params["fc1"], x))
    x = jax.nn.relu(_linear(params["fc2"], x))
    x = jax.nn.relu(_linear(params["fc3"], x))
    mu = jnp.mean(x, axis=0)
    var = jnp.var(x, axis=0)
    x = params["bn_gamma"] * (x - mu) / jnp.sqrt(var + 1e-5) + params["bn_beta"]
    return jax.nn.relu(_linear(params["fc4"], x))
